# baseline (speedup 1.0000x reference)
import functools

import jax
import jax.numpy as jnp
from jax import lax
from jax.experimental import pallas as pl
from jax.experimental.pallas import tpu as pltpu
from jax.experimental.pallas import tpu_sc as plsc

B, N, F = 16, 576, 768
H, M = 12, 1024
D = F // H
BN = B * N
HBN = BN // 2
TN = 768
GRID = HBN // TN
EXPIRE_THRESHOLD = 0.05

NC, NS = 2, 16
NW = NC * NS
CHUNKS_PER_HEAD = 8
TASK_ROWS = HBN // CHUNKS_PER_HEAD
TASKS_PER_WORKER = (H * CHUNKS_PER_HEAD) // NW
GSZ = 96
GATHERS_PER_TASK = TASK_ROWS // GSZ


def _init_c2(cbT_ref, c2_s):
    for h in range(H):
        cTn = cbT_ref[h]
        c2_s[h:h + 1, :] = 0.25 * jnp.sum(cTn * cTn, axis=0, keepdims=True)


def _head_loop(x_ref, cbT_ref, gidx_ref, counts_s, loss_s, c2_s, idxc_s):
    xb = x_ref[...]
    ones_row = jnp.ones((1, TN), dtype=jnp.float32)
    iota = jax.lax.broadcasted_iota(jnp.int32, (TN, M), 1)
    loss_acc = jnp.float32(0.0)
    for h in range(H):
        q = jax.lax.slice(xb, (0, h * D), (TN, (h + 1) * D))
        cT = cbT_ref[h]
        qc = jax.lax.dot_general(q, cT, (((1,), (0,)), ((), ())),
                                 preferred_element_type=jnp.float32)
        c2 = c2_s[h:h + 1, :]
        s = c2 + qc
        minv = jnp.min(s, axis=1, keepdims=True)
        eqm = s == minv
        sel = jnp.where(eqm, iota, M)
        idxh = jnp.min(sel, axis=1, keepdims=True)
        idxc_s[:, h:h + 1] = idxh + jnp.int32(h * M)
        ohf = eqm.astype(jnp.float32)
        cnt = jax.lax.dot_general(ones_row, ohf, (((1,), (0,)), ((), ())),
                                  preferred_element_type=jnp.float32)
        counts_s[h:h + 1, :] = counts_s[h:h + 1, :] + cnt
        q2 = jnp.sum(q * q, axis=1, keepdims=True)
        loss_acc = loss_acc + jnp.sum(jnp.maximum(q2 + minv, 0.0))

    loss_s[0] = loss_s[0] + loss_acc
    gidx_ref[...] = jax.lax.transpose(idxc_s[...], (1, 0))


def _vq_body_a(x_ref, cbT_ref,
               gidx_ref, counts_ref, lsum_ref,
               counts_s, loss_s, c2_s, idxc_s):
    i = pl.program_id(0)

    @pl.when(i == 0)
    def _init():
        counts_s[...] = jnp.zeros_like(counts_s)
        loss_s[0] = jnp.float32(0.0)
        _init_c2(cbT_ref, c2_s)

    _head_loop(x_ref, cbT_ref, gidx_ref, counts_s, loss_s, c2_s, idxc_s)

    @pl.when(i == GRID - 1)
    def _fin():
        counts_ref[...] = counts_s[...]
        lsum_ref[0] = loss_s[0]


def _vq_body_b(x_ref, cbT_ref, be_ref, cin_ref, lin_ref,
               gidx_ref, loss_ref, perp_ref, repl_ref,
               counts_s, loss_s, c2_s, idxc_s):
    i = pl.program_id(0)

    @pl.when(i == 0)
    def _init():
        counts_s[...] = cin_ref[...]
        loss_s[0] = lin_ref[0]
        _init_c2(cbT_ref, c2_s)

    _head_loop(x_ref, cbT_ref, gidx_ref, counts_s, loss_s, c2_s, idxc_s)

    @pl.when(i == GRID - 1)
    def _fin():
        loss_ref[0, 0] = loss_s[0] / jnp.float32(BN * F)
        mean = counts_s[...] * jnp.float32(1.0 / BN)
        ent = jnp.sum(mean * jnp.log(mean + 1e-10), axis=1)
        perp_ref[...] = jnp.broadcast_to(jnp.exp(-ent)[:, None], (H, 128))
        expired = (be_ref[...] < EXPIRE_THRESHOLD).astype(jnp.int32)
        repl_ref[...] = jnp.broadcast_to(jnp.sum(expired, axis=1)[:, None],
                                         (H, 128))


_sc_mesh = plsc.VectorSubcoreMesh(core_axis_name="c", subcore_axis_name="s")


@functools.partial(
    pl.kernel,
    mesh=_sc_mesh,
    out_type=jax.ShapeDtypeStruct((HBN, H, D), jnp.float32),
    scratch_types=[
        pltpu.VMEM((TASK_ROWS,), jnp.int32),
        pltpu.VMEM((TASK_ROWS,), jnp.int32),
        pltpu.VMEM((TASK_ROWS, D), jnp.float32),
        pltpu.VMEM((TASK_ROWS, D), jnp.float32),
        pltpu.SemaphoreType.DMA,
        pltpu.SemaphoreType.DMA,
    ],
    compiler_params=pltpu.CompilerParams(use_tc_tiling_on_sc=False),
)
def _sc_gather(cb_hbm, gidx_hbm, out_hbm, idx_v0, idx_v1, rows_v0, rows_v1,
               gsem, wsem):
    wid = lax.axis_index("s") * NC + lax.axis_index("c")
    idx_bufs = [idx_v0, idx_v1]
    row_bufs = [rows_v0, rows_v1]

    def _hc(t):
        task = wid * TASKS_PER_WORKER + t
        return task // CHUNKS_PER_HEAD, task % CHUNKS_PER_HEAD

    h0, c0 = _hc(0)
    pltpu.sync_copy(gidx_hbm.at[h0, pl.ds(c0 * TASK_ROWS, TASK_ROWS)], idx_v0)
    write_handles = [None, None]
    for t in range(TASKS_PER_WORKER):
        ib = idx_bufs[t % 2]
        rb = row_bufs[t % 2]
        h, ch = _hc(t)
        wh = write_handles[t % 2]
        if wh is not None:
            wh.wait()
            write_handles[t % 2] = None
        handles = []
        for j in range(GATHERS_PER_TASK):
            handles.append(pltpu.async_copy(
                cb_hbm.at[ib.at[pl.ds(j * GSZ, GSZ)]],
                rb.at[pl.ds(j * GSZ, GSZ)],
                gsem))
        if t + 1 < TASKS_PER_WORKER:
            hn, cn = _hc(t + 1)
            pltpu.sync_copy(
                gidx_hbm.at[hn, pl.ds(cn * TASK_ROWS, TASK_ROWS)],
                idx_bufs[(t + 1) % 2])
        for hd in handles:
            hd.wait()
        write_handles[t % 2] = pltpu.async_copy(
            rb, out_hbm.at[pl.ds(ch * TASK_ROWS, TASK_ROWS), h], wsem)
    for wh in write_handles:
        if wh is not None:
            wh.wait()


def kernel(x, codebooks, budget_ema):
    xr = x.reshape(BN, F)
    cbT = codebooks.transpose(0, 2, 1) * jnp.float32(-2.0)
    cb_flat = codebooks.reshape(H * M, D)

    common = dict(
        grid=(GRID,),
        compiler_params=pltpu.CompilerParams(
            dimension_semantics=("arbitrary",),
        ),
    )
    x_spec = pl.BlockSpec((TN, F), lambda i: (i, 0))
    cb_spec = pl.BlockSpec((H, D, M), lambda i: (0, 0, 0))
    full_spec = lambda r, c: pl.BlockSpec((r, c), lambda i: (0, 0))
    gidx_spec = pl.BlockSpec((16, TN), lambda i: (0, i))
    scratches = [
        pltpu.VMEM((H, M), jnp.float32),
        pltpu.SMEM((1,), jnp.float32),
        pltpu.VMEM((H, M), jnp.float32),
        pltpu.VMEM((TN, 16), jnp.int32),
    ]

    gidx_a, counts_a, lsum_a = pl.pallas_call(
        _vq_body_a,
        in_specs=[x_spec, cb_spec],
        out_specs=[gidx_spec, full_spec(H, M),
                   pl.BlockSpec(memory_space=pltpu.SMEM)],
        out_shape=[
            jax.ShapeDtypeStruct((16, HBN), jnp.int32),
            jax.ShapeDtypeStruct((H, M), jnp.float32),
            jax.ShapeDtypeStruct((1,), jnp.float32),
        ],
        scratch_shapes=scratches,
        **common,
    )(xr[:HBN], cbT)

    out_a = _sc_gather(cb_flat, gidx_a)

    gidx_b, loss_o, perp_o, repl_o = pl.pallas_call(
        _vq_body_b,
        in_specs=[x_spec, cb_spec, full_spec(H, M), full_spec(H, M),
                  pl.BlockSpec(memory_space=pltpu.SMEM)],
        out_specs=[gidx_spec,
                   pl.BlockSpec(memory_space=pltpu.SMEM),
                   full_spec(H, 128), full_spec(H, 128)],
        out_shape=[
            jax.ShapeDtypeStruct((16, HBN), jnp.int32),
            jax.ShapeDtypeStruct((1, 1), jnp.float32),
            jax.ShapeDtypeStruct((H, 128), jnp.float32),
            jax.ShapeDtypeStruct((H, 128), jnp.int32),
        ],
        scratch_shapes=scratches,
        **common,
    )(xr[HBN:], cbT, budget_ema, counts_a, lsum_a)

    out_b = _sc_gather(cb_flat, gidx_b)

    out = jnp.concatenate([out_a, out_b], axis=0).reshape(B, N, F)
    offs = (jnp.arange(H, dtype=jnp.int32) * M)[:, None]
    idx_full = jnp.concatenate([gidx_a[:H, :] - offs, gidx_b[:H, :] - offs],
                               axis=1)
    codebook_indices = idx_full.reshape(H, B, N).transpose(1, 0, 2)
    loss = loss_o[0, 0]
    perp = perp_o[:, 0]
    replaced_codes = repl_o[:, 0]
    return (out, codebook_indices, loss, perp, replaced_codes, budget_ema)


# speedup vs baseline: 1.5076x; 1.5076x over previous
import functools

import jax
import jax.numpy as jnp
from jax import lax
from jax.experimental import pallas as pl
from jax.experimental.pallas import tpu as pltpu
from jax.experimental.pallas import tpu_sc as plsc

B, N, F = 16, 576, 768
H, M = 12, 1024
D = F // H
BN = B * N
TN = 768
GRID = BN // TN
EXPIRE_THRESHOLD = 0.05

NC, NS = 2, 16
NW = NC * NS
CHUNKS_PER_HEAD = 16
TASK_ROWS = BN // CHUNKS_PER_HEAD
TASKS_PER_WORKER = (H * CHUNKS_PER_HEAD) // NW
GSZ = 96
GATHERS_PER_TASK = TASK_ROWS // GSZ


def _vq_body(x_ref, cbT_ref, be_ref,
             gidx_ref, loss_ref, perp_ref, repl_ref,
             counts_s, loss_s, c2_s, idxc_s):
    i = pl.program_id(0)

    @pl.when(i == 0)
    def _init():
        counts_s[...] = jnp.zeros_like(counts_s)
        loss_s[0] = jnp.float32(0.0)
        for h in range(H):
            cTn = cbT_ref[h]
            c2_s[h:h + 1, :] = 0.25 * jnp.sum(cTn * cTn, axis=0,
                                              keepdims=True)
    xb = x_ref[...]
    ones_row = jnp.ones((1, TN), dtype=jnp.float32)
    iota = jax.lax.broadcasted_iota(jnp.int32, (TN, M), 1)
    loss_acc = jnp.float32(0.0)
    for h in range(H):
        q = jax.lax.slice(xb, (0, h * D), (TN, (h + 1) * D))
        cT = cbT_ref[h]
        qc = jax.lax.dot_general(q, cT, (((1,), (0,)), ((), ())),
                                 preferred_element_type=jnp.float32)
        c2 = c2_s[h:h + 1, :]
        s = c2 + qc
        minv = jnp.min(s, axis=1, keepdims=True)
        eqm = s == minv
        sel = jnp.where(eqm, iota, M)
        idxh = jnp.min(sel, axis=1, keepdims=True)
        idxc_s[:, h:h + 1] = idxh + jnp.int32(h * M)
        ohf = eqm.astype(jnp.float32)
        cnt = jax.lax.dot_general(ones_row, ohf, (((1,), (0,)), ((), ())),
                                  preferred_element_type=jnp.float32)
        counts_s[h:h + 1, :] = counts_s[h:h + 1, :] + cnt
        q2 = jnp.sum(q * q, axis=1, keepdims=True)
        loss_acc = loss_acc + jnp.sum(jnp.maximum(q2 + minv, 0.0))

    loss_s[0] = loss_s[0] + loss_acc
    gidx_ref[...] = jax.lax.transpose(idxc_s[...], (1, 0))

    @pl.when(i == GRID - 1)
    def _fin():
        loss_ref[0, 0] = loss_s[0] / jnp.float32(BN * F)
        mean = counts_s[...] * jnp.float32(1.0 / BN)
        ent = jnp.sum(mean * jnp.log(mean + 1e-10), axis=1)
        perp_ref[...] = jnp.broadcast_to(jnp.exp(-ent)[:, None], (H, 128))
        expired = (be_ref[...] < EXPIRE_THRESHOLD).astype(jnp.int32)
        repl_ref[...] = jnp.broadcast_to(jnp.sum(expired, axis=1)[:, None],
                                         (H, 128))


_sc_mesh = plsc.VectorSubcoreMesh(core_axis_name="c", subcore_axis_name="s")


@functools.partial(
    pl.kernel,
    mesh=_sc_mesh,
    out_type=jax.ShapeDtypeStruct((BN, H, D), jnp.float32),
    scratch_types=[
        pltpu.VMEM((TASK_ROWS,), jnp.int32),
        pltpu.VMEM((TASK_ROWS,), jnp.int32),
        pltpu.VMEM((TASK_ROWS, D), jnp.float32),
        pltpu.VMEM((TASK_ROWS, D), jnp.float32),
        pltpu.SemaphoreType.DMA,
        pltpu.SemaphoreType.DMA,
    ],
    compiler_params=pltpu.CompilerParams(use_tc_tiling_on_sc=False),
)
def _sc_gather(cb_hbm, gidx_hbm, out_hbm, idx_v0, idx_v1, rows_v0, rows_v1,
               gsem, wsem):
    wid = lax.axis_index("s") * NC + lax.axis_index("c")
    idx_bufs = [idx_v0, idx_v1]
    row_bufs = [rows_v0, rows_v1]

    def _hc(t):
        task = wid * TASKS_PER_WORKER + t
        return task // CHUNKS_PER_HEAD, task % CHUNKS_PER_HEAD

    h0, c0 = _hc(0)
    pltpu.sync_copy(gidx_hbm.at[h0, pl.ds(c0 * TASK_ROWS, TASK_ROWS)], idx_v0)
    write_handles = [None, None]
    for t in range(TASKS_PER_WORKER):
        ib = idx_bufs[t % 2]
        rb = row_bufs[t % 2]
        h, ch = _hc(t)
        wh = write_handles[t % 2]
        if wh is not None:
            wh.wait()
            write_handles[t % 2] = None
        handles = []
        for j in range(GATHERS_PER_TASK):
            handles.append(pltpu.async_copy(
                cb_hbm.at[ib.at[pl.ds(j * GSZ, GSZ)]],
                rb.at[pl.ds(j * GSZ, GSZ)],
                gsem))
        if t + 1 < TASKS_PER_WORKER:
            hn, cn = _hc(t + 1)
            pltpu.sync_copy(
                gidx_hbm.at[hn, pl.ds(cn * TASK_ROWS, TASK_ROWS)],
                idx_bufs[(t + 1) % 2])
        for hd in handles:
            hd.wait()
        write_handles[t % 2] = pltpu.async_copy(
            rb, out_hbm.at[pl.ds(ch * TASK_ROWS, TASK_ROWS), h], wsem)
    for wh in write_handles:
        if wh is not None:
            wh.wait()


def kernel(x, codebooks, budget_ema):
    xr = x.reshape(BN, F)
    cbT = codebooks.transpose(0, 2, 1) * jnp.float32(-2.0)

    gidx_o, loss_o, perp_o, repl_o = pl.pallas_call(
        _vq_body,
        grid=(GRID,),
        in_specs=[
            pl.BlockSpec((TN, F), lambda i: (i, 0)),
            pl.BlockSpec((H, D, M), lambda i: (0, 0, 0)),
            pl.BlockSpec((H, M), lambda i: (0, 0)),
        ],
        out_specs=[
            pl.BlockSpec((16, TN), lambda i: (0, i)),
            pl.BlockSpec(memory_space=pltpu.SMEM),
            pl.BlockSpec((H, 128), lambda i: (0, 0)),
            pl.BlockSpec((H, 128), lambda i: (0, 0)),
        ],
        out_shape=[
            jax.ShapeDtypeStruct((16, BN), jnp.int32),
            jax.ShapeDtypeStruct((1, 1), jnp.float32),
            jax.ShapeDtypeStruct((H, 128), jnp.float32),
            jax.ShapeDtypeStruct((H, 128), jnp.int32),
        ],
        scratch_shapes=[
            pltpu.VMEM((H, M), jnp.float32),
            pltpu.SMEM((1,), jnp.float32),
            pltpu.VMEM((H, M), jnp.float32),
            pltpu.VMEM((TN, 16), jnp.int32),
        ],
        compiler_params=pltpu.CompilerParams(
            dimension_semantics=("arbitrary",),
        ),
    )(xr, cbT, budget_ema)

    cb_flat = codebooks.reshape(H * M, D)
    out_rhd = _sc_gather(cb_flat, gidx_o)

    out = out_rhd.reshape(B, N, F)
    codebook_indices = (gidx_o[:H, :]
                        - (jnp.arange(H, dtype=jnp.int32) * M)[:, None]
                        ).reshape(H, B, N).transpose(1, 0, 2)
    loss = loss_o[0, 0]
    perp = perp_o[:, 0]
    replaced_codes = repl_o[:, 0]
    return (out, codebook_indices, loss, perp, replaced_codes, budget_ema)
